# initial kernel scaffold (unmeasured)
import jax
import jax.numpy as jnp
from jax import lax
from jax.experimental import pallas as pl
from jax.experimental.pallas import tpu as pltpu

N_DEV = 16


def kernel(x, w_mat):
    m, _ = x.shape
    _, n = w_mat.shape
    chunk = m // N_DEV

    def body(x_ref, w_ref, out_ref, rs_buf, ag_buf,
             rs_send, rs_recv, ag_send, ag_recv):
        my = lax.axis_index("i")
        right = (my + 1) % N_DEV

        out_ref[:, :] = jnp.dot(
            x_ref[:, :], w_ref[:, :], preferred_element_type=jnp.float32
        )

        for s in range(N_DEV - 1):
            sc = (my + (N_DEV - s)) % N_DEV
            rc = (my + (N_DEV - s - 1)) % N_DEV
            rdma = pltpu.make_async_remote_copy(
                src_ref=out_ref.at[pl.ds(sc * chunk, chunk), :],
                dst_ref=rs_buf.at[s],
                send_sem=rs_send.at[s],
                recv_sem=rs_recv.at[s],
                device_id=(right,),
                device_id_type=pl.DeviceIdType.MESH,
            )
            rdma.start()
            rdma.wait()
            out_ref[pl.ds(rc * chunk, chunk), :] += rs_buf[s]

        own = (my + 1) % N_DEV
        y = out_ref[pl.ds(own * chunk, chunk), :]
        out_ref[pl.ds(own * chunk, chunk), :] = y * jax.nn.sigmoid(y)

        for h in range(N_DEV - 1):
            sc = (my + (N_DEV + 1 - h)) % N_DEV
            rc = (my + (N_DEV - h)) % N_DEV
            rdma = pltpu.make_async_remote_copy(
                src_ref=out_ref.at[pl.ds(sc * chunk, chunk), :],
                dst_ref=ag_buf.at[h],
                send_sem=ag_send.at[h],
                recv_sem=ag_recv.at[h],
                device_id=(right,),
                device_id_type=pl.DeviceIdType.MESH,
            )
            rdma.start()
            rdma.wait()
            out_ref[pl.ds(rc * chunk, chunk), :] = ag_buf[h]

    return pl.pallas_call(
        body,
        out_shape=jax.ShapeDtypeStruct((m, n), jnp.float32),
        in_specs=[
            pl.BlockSpec(memory_space=pltpu.VMEM),
            pl.BlockSpec(memory_space=pltpu.VMEM),
        ],
        out_specs=pl.BlockSpec(memory_space=pltpu.VMEM),
        scratch_shapes=[
            pltpu.VMEM((N_DEV - 1, chunk, n), jnp.float32),
            pltpu.VMEM((N_DEV - 1, chunk, n), jnp.float32),
            pltpu.SemaphoreType.DMA((N_DEV - 1,)),
            pltpu.SemaphoreType.DMA((N_DEV - 1,)),
            pltpu.SemaphoreType.DMA((N_DEV - 1,)),
            pltpu.SemaphoreType.DMA((N_DEV - 1,)),
        ],
        compiler_params=pltpu.CompilerParams(collective_id=0),
    )(x, w_mat)


# baseline (device time: 154179 ns/iter reference)
import jax
import jax.numpy as jnp
from jax import lax
from jax.experimental import pallas as pl
from jax.experimental.pallas import tpu as pltpu

N_DEV = 16


def kernel(x, w_mat):
    m, _ = x.shape
    _, n = w_mat.shape
    chunk = m // N_DEV

    def body(x_ref, w_ref, out_ref, rs_buf, ag_buf,
             rs_send, rs_recv, ag_send, ag_recv):
        my = lax.axis_index("i")
        right = (my + 1) % N_DEV

        out_ref[:, :] = jnp.dot(
            x_ref[:, :], w_ref[:, :], preferred_element_type=jnp.float32
        )

        for s in range(N_DEV - 1):
            sc = (my + (N_DEV - s)) % N_DEV
            rc = (my + (N_DEV - s - 1)) % N_DEV
            rdma = pltpu.make_async_remote_copy(
                src_ref=out_ref.at[pl.ds(sc * chunk, chunk), :],
                dst_ref=rs_buf.at[s],
                send_sem=rs_send.at[s],
                recv_sem=rs_recv.at[s],
                device_id=(right,),
                device_id_type=pl.DeviceIdType.MESH,
            )
            rdma.start()
            rdma.wait()
            out_ref[pl.ds(rc * chunk, chunk), :] += rs_buf[s]

        own = (my + 1) % N_DEV
        y = out_ref[pl.ds(own * chunk, chunk), :]
        out_ref[pl.ds(own * chunk, chunk), :] = y * jax.nn.sigmoid(y)

        for h in range(N_DEV - 1):
            sc = (my + (N_DEV + 1 - h)) % N_DEV
            rc = (my + (N_DEV - h)) % N_DEV
            rdma = pltpu.make_async_remote_copy(
                src_ref=out_ref.at[pl.ds(sc * chunk, chunk), :],
                dst_ref=ag_buf.at[h],
                send_sem=ag_send.at[h],
                recv_sem=ag_recv.at[h],
                device_id=(right,),
                device_id_type=pl.DeviceIdType.MESH,
            )
            rdma.start()
            rdma.wait()
            out_ref[pl.ds(rc * chunk, chunk), :] = ag_buf[h]

    return pl.pallas_call(
        body,
        out_shape=jax.ShapeDtypeStruct((m, n), jnp.float32),
        in_specs=[
            pl.BlockSpec(memory_space=pltpu.VMEM),
            pl.BlockSpec(memory_space=pltpu.VMEM),
        ],
        out_specs=pl.BlockSpec(memory_space=pltpu.VMEM),
        scratch_shapes=[
            pltpu.VMEM((N_DEV - 1, chunk, n), jnp.float32),
            pltpu.VMEM((N_DEV - 1, chunk, n), jnp.float32),
            pltpu.SemaphoreType.DMA((N_DEV - 1,)),
            pltpu.SemaphoreType.DMA((N_DEV - 1,)),
            pltpu.SemaphoreType.DMA((N_DEV - 1,)),
            pltpu.SemaphoreType.DMA((N_DEV - 1,)),
        ],
    )(x, w_mat)


# device time: 86678 ns/iter; 1.7788x vs baseline; 1.7788x over previous
import jax
import jax.numpy as jnp
from jax import lax
from jax.experimental import pallas as pl
from jax.experimental.pallas import tpu as pltpu

N_DEV = 16
C = 4


def kernel(x, w_mat):
    m, _ = x.shape
    _, n = w_mat.shape
    w = n // C
    halves = [m // 2, m // 4, m // 8, m // 16]

    def body(x_ref, w_ref, out_ref, b0, b1, b2, b3, send_sems, recv_sems):
        my = lax.axis_index("i")
        z = my // 4
        q = my % 4
        my_x = (q ^ (q >> 1)) & 1
        my_y = q >> 1
        bits = [my_x, my_y, z & 1, (z >> 1) & 1]
        partners = [
            4 * z + (q ^ 1),
            4 * z + (q ^ 3),
            4 * (z ^ 1) + q,
            4 * (z ^ 2) + q,
        ]
        bufs = [b0, b1, b2, b3]

        out_ref[:, :] = jnp.dot(
            x_ref[:, :], w_ref[:, :], preferred_element_type=jnp.float32
        )

        def rs_rdma(k, c, base):
            half = halves[k]
            pbase = base + (1 - bits[k]) * half
            return pltpu.make_async_remote_copy(
                src_ref=out_ref.at[pl.ds(pbase, half), pl.ds(c * w, w)],
                dst_ref=bufs[k].at[c],
                send_sem=send_sems.at[k, c],
                recv_sem=recv_sems.at[k, c],
                device_id=(partners[k],),
                device_id_type=pl.DeviceIdType.MESH,
            )

        def ag_rdma(j, c, base, size):
            return pltpu.make_async_remote_copy(
                src_ref=out_ref.at[pl.ds(base, size), pl.ds(c * w, w)],
                dst_ref=out_ref.at[pl.ds(base, size), pl.ds(c * w, w)],
                send_sem=send_sems.at[4 + j, c],
                recv_sem=recv_sems.at[4 + j, c],
                device_id=(partners[3 - j],),
                device_id_type=pl.DeviceIdType.MESH,
            )

        rs = {}
        ag = {}
        for c in range(C):
            rs[(0, c)] = rs_rdma(0, c, 0)
            rs[(0, c)].start()

        bases = [0]
        for k in range(4):
            half = halves[k]
            mbase = bases[k] + bits[k] * half
            for c in range(C):
                rs[(k, c)].wait()
                out_ref[pl.ds(mbase, half), pl.ds(c * w, w)] += bufs[k][c]
                if k < 3:
                    rs[(k + 1, c)] = rs_rdma(k + 1, c, mbase)
                    rs[(k + 1, c)].start()
                else:
                    yv = out_ref[pl.ds(mbase, half), pl.ds(c * w, w)]
                    out_ref[pl.ds(mbase, half), pl.ds(c * w, w)] = (
                        yv * jax.nn.sigmoid(yv)
                    )
                    ag[(0, c)] = ag_rdma(0, c, mbase, half)
                    ag[(0, c)].start()
            bases.append(mbase)

        for j in range(4):
            size = halves[3 - j]
            for c in range(C):
                ag[(j, c)].wait()
                if j < 3:
                    ag[(j + 1, c)] = ag_rdma(j + 1, c, bases[3 - j], 2 * size)
                    ag[(j + 1, c)].start()

    return pl.pallas_call(
        body,
        out_shape=jax.ShapeDtypeStruct((m, n), jnp.float32),
        in_specs=[
            pl.BlockSpec(memory_space=pltpu.VMEM),
            pl.BlockSpec(memory_space=pltpu.VMEM),
        ],
        out_specs=pl.BlockSpec(memory_space=pltpu.VMEM),
        scratch_shapes=[
            pltpu.VMEM((C, halves[0], w), jnp.float32),
            pltpu.VMEM((C, halves[1], w), jnp.float32),
            pltpu.VMEM((C, halves[2], w), jnp.float32),
            pltpu.VMEM((C, halves[3], w), jnp.float32),
            pltpu.SemaphoreType.DMA((8, C)),
            pltpu.SemaphoreType.DMA((8, C)),
        ],
    )(x, w_mat)


# device time: 72410 ns/iter; 2.1293x vs baseline; 1.1970x over previous
import jax
import jax.numpy as jnp
from jax import lax
from jax.experimental import pallas as pl
from jax.experimental.pallas import tpu as pltpu

N_DEV = 16
ORDERS = ((0, 1, 2, 3), (1, 0, 2, 3))
C2 = 2


def kernel(x, w_mat):
    m, _ = x.shape
    _, n = w_mat.shape
    n_parts = len(ORDERS)
    pcols = n // n_parts
    wch = pcols // C2
    halves = [m >> (k + 1) for k in range(4)]

    def body(x_ref, w_ref, out_ref, b0, b1, b2, b3, send_sems, recv_sems):
        my = lax.axis_index("i")
        z = my // 4
        q = my % 4
        bits = [(q ^ (q >> 1)) & 1, q >> 1, z & 1, (z >> 1) & 1]
        partners = [
            4 * z + (q ^ 1),
            4 * z + (q ^ 3),
            4 * (z ^ 1) + q,
            4 * (z ^ 2) + q,
        ]
        bufs = [b0, b1, b2, b3]

        out_ref[:, :] = jnp.dot(
            x_ref[:, :], w_ref[:, :], preferred_element_type=jnp.float32
        )

        def cols(p, c):
            return pl.ds(p * pcols + c * wch, wch)

        def rs_rdma(p, k, c, base):
            dim = ORDERS[p][k]
            half = halves[k]
            pbase = base + (1 - bits[dim]) * half
            return pltpu.make_async_remote_copy(
                src_ref=out_ref.at[pl.ds(pbase, half), cols(p, c)],
                dst_ref=bufs[k].at[p * C2 + c],
                send_sem=send_sems.at[k, p * C2 + c],
                recv_sem=recv_sems.at[k, p * C2 + c],
                device_id=(partners[dim],),
                device_id_type=pl.DeviceIdType.MESH,
            )

        def ag_rdma(p, j, c, base, size):
            dim = ORDERS[p][3 - j]
            return pltpu.make_async_remote_copy(
                src_ref=out_ref.at[pl.ds(base, size), cols(p, c)],
                dst_ref=out_ref.at[pl.ds(base, size), cols(p, c)],
                send_sem=send_sems.at[4 + j, p * C2 + c],
                recv_sem=recv_sems.at[4 + j, p * C2 + c],
                device_id=(partners[dim],),
                device_id_type=pl.DeviceIdType.MESH,
            )

        rs = {}
        ag = {}
        for p in range(n_parts):
            for c in range(C2):
                rs[(p, 0, c)] = rs_rdma(p, 0, c, 0)
                rs[(p, 0, c)].start()

        bases = [[0] for _ in range(n_parts)]
        for k in range(4):
            half = halves[k]
            for p in range(n_parts):
                mbase = bases[p][k] + bits[ORDERS[p][k]] * half
                for c in range(C2):
                    rs[(p, k, c)].wait()
                    out_ref[pl.ds(mbase, half), cols(p, c)] += bufs[k][p * C2 + c]
                    if k < 3:
                        rs[(p, k + 1, c)] = rs_rdma(p, k + 1, c, mbase)
                        rs[(p, k + 1, c)].start()
                    else:
                        yv = out_ref[pl.ds(mbase, half), cols(p, c)]
                        out_ref[pl.ds(mbase, half), cols(p, c)] = (
                            yv * jax.nn.sigmoid(yv)
                        )
                        ag[(p, 0, c)] = ag_rdma(p, 0, c, mbase, half)
                        ag[(p, 0, c)].start()
                bases[p].append(mbase)

        for j in range(4):
            size = halves[3 - j]
            for p in range(n_parts):
                for c in range(C2):
                    ag[(p, j, c)].wait()
                    if j < 3:
                        ag[(p, j + 1, c)] = ag_rdma(
                            p, j + 1, c, bases[p][3 - j], 2 * size
                        )
                        ag[(p, j + 1, c)].start()

    nbuf = n_parts * C2
    return pl.pallas_call(
        body,
        out_shape=jax.ShapeDtypeStruct((m, n), jnp.float32),
        in_specs=[
            pl.BlockSpec(memory_space=pltpu.VMEM),
            pl.BlockSpec(memory_space=pltpu.VMEM),
        ],
        out_specs=pl.BlockSpec(memory_space=pltpu.VMEM),
        scratch_shapes=[
            pltpu.VMEM((nbuf, halves[0], wch), jnp.float32),
            pltpu.VMEM((nbuf, halves[1], wch), jnp.float32),
            pltpu.VMEM((nbuf, halves[2], wch), jnp.float32),
            pltpu.VMEM((nbuf, halves[3], wch), jnp.float32),
            pltpu.SemaphoreType.DMA((8, nbuf)),
            pltpu.SemaphoreType.DMA((8, nbuf)),
        ],
    )(x, w_mat)


# device time: 65821 ns/iter; 2.3424x vs baseline; 1.1001x over previous
import jax
import jax.numpy as jnp
from jax import lax
from jax.experimental import pallas as pl
from jax.experimental.pallas import tpu as pltpu

N_DEV = 16
ORDERS = ((0, 1), (1, 0))
C2 = 4


def kernel(x, w_mat):
    m, _ = x.shape
    _, n = w_mat.shape
    n_parts = len(ORDERS)
    pcols = n // n_parts
    wch = pcols // C2
    nu = n_parts * C2
    h0, h1 = m // 2, m // 4
    zrows = m // N_DEV

    def body(x_ref, w_ref, out_ref, b0, b1, zbuf,
             rs_send, rs_recv, zs_send, zs_recv, zb_send, zb_recv,
             ag_send, ag_recv):
        my = lax.axis_index("i")
        z = my // 4
        q = my % 4
        bits = [(q ^ (q >> 1)) & 1, q >> 1]
        partners = [
            4 * z + (q ^ 1),
            4 * z + (q ^ 3),
        ]
        zpeers = [4 * ((z + d) % 4) + q for d in (1, 2, 3)]

        barrier = pltpu.get_barrier_semaphore()
        for d in (partners[0], partners[1], zpeers[0], zpeers[1], zpeers[2]):
            pl.semaphore_signal(
                barrier, inc=1, device_id=(d,),
                device_id_type=pl.DeviceIdType.MESH,
            )
        pl.semaphore_wait(barrier, 5)

        out_ref[:, :] = jnp.dot(
            x_ref[:, :], w_ref[:, :], preferred_element_type=jnp.float32
        )

        def cols(p, c):
            return pl.ds(p * pcols + c * wch, wch)

        def unit(p, c):
            return p * C2 + c

        def rs_rdma(p, k, c, base):
            half = (h0, h1)[k]
            dim = ORDERS[p][k]
            pbase = base + (1 - bits[dim]) * half
            return pltpu.make_async_remote_copy(
                src_ref=out_ref.at[pl.ds(pbase, half), cols(p, c)],
                dst_ref=(b0, b1)[k].at[unit(p, c)],
                send_sem=rs_send.at[k, unit(p, c)],
                recv_sem=rs_recv.at[k, unit(p, c)],
                device_id=(partners[dim],),
                device_id_type=pl.DeviceIdType.MESH,
            )

        def zscatter_rdma(p, c, t, base):
            tz = (z + t + 1) % 4
            slot = t
            return pltpu.make_async_remote_copy(
                src_ref=out_ref.at[pl.ds(base + tz * zrows, zrows), cols(p, c)],
                dst_ref=zbuf.at[unit(p, c), slot],
                send_sem=zs_send.at[unit(p, c), t],
                recv_sem=zs_recv.at[unit(p, c), slot],
                device_id=(zpeers[t],),
                device_id_type=pl.DeviceIdType.MESH,
            )

        def zbcast_rdma(p, c, t, fbase):
            return pltpu.make_async_remote_copy(
                src_ref=out_ref.at[pl.ds(fbase, zrows), cols(p, c)],
                dst_ref=out_ref.at[pl.ds(fbase, zrows), cols(p, c)],
                send_sem=zb_send.at[unit(p, c), t],
                recv_sem=zb_recv.at[unit(p, c), t],
                device_id=(zpeers[t],),
                device_id_type=pl.DeviceIdType.MESH,
            )

        def ag_rdma(p, j, c, base, size):
            dim = ORDERS[p][1 - j]
            return pltpu.make_async_remote_copy(
                src_ref=out_ref.at[pl.ds(base, size), cols(p, c)],
                dst_ref=out_ref.at[pl.ds(base, size), cols(p, c)],
                send_sem=ag_send.at[j, unit(p, c)],
                recv_sem=ag_recv.at[j, unit(p, c)],
                device_id=(partners[dim],),
                device_id_type=pl.DeviceIdType.MESH,
            )

        rs, zs, zb, ag = {}, {}, {}, {}
        for p in range(n_parts):
            for c in range(C2):
                rs[(p, 0, c)] = rs_rdma(p, 0, c, 0)
                rs[(p, 0, c)].start()

        bases = [[0] for _ in range(n_parts)]
        for k in range(2):
            half = (h0, h1)[k]
            for p in range(n_parts):
                mbase = bases[p][k] + bits[ORDERS[p][k]] * half
                for c in range(C2):
                    rs[(p, k, c)].wait()
                    out_ref[pl.ds(mbase, half), cols(p, c)] += (
                        (b0, b1)[k][unit(p, c)]
                    )
                    if k == 0:
                        rs[(p, 1, c)] = rs_rdma(p, 1, c, mbase)
                        rs[(p, 1, c)].start()
                    else:
                        for t in range(3):
                            zs[(p, c, t)] = zscatter_rdma(p, c, t, mbase)
                            zs[(p, c, t)].start()
                bases[p].append(mbase)

        for p in range(n_parts):
            fbase = bases[p][2] + z * zrows
            for c in range(C2):
                for t in range(3):
                    zs[(p, c, t)].wait_recv()
                u = unit(p, c)
                yv = (
                    out_ref[pl.ds(fbase, zrows), cols(p, c)]
                    + zbuf[u, 0] + zbuf[u, 1] + zbuf[u, 2]
                )
                out_ref[pl.ds(fbase, zrows), cols(p, c)] = (
                    yv * jax.nn.sigmoid(yv)
                )
                for t in range(3):
                    zb[(p, c, t)] = zbcast_rdma(p, c, t, fbase)
                    zb[(p, c, t)].start()
                for t in range(3):
                    zs[(p, c, t)].wait_send()

        for p in range(n_parts):
            for c in range(C2):
                for t in range(3):
                    zb[(p, c, t)].wait_recv()
                ag[(p, 0, c)] = ag_rdma(p, 0, c, bases[p][2], h1)
                ag[(p, 0, c)].start()
                for t in range(3):
                    zb[(p, c, t)].wait_send()

        for j in range(2):
            for p in range(n_parts):
                for c in range(C2):
                    ag[(p, j, c)].wait()
                    if j == 0:
                        ag[(p, 1, c)] = ag_rdma(p, 1, c, bases[p][1], h0)
                        ag[(p, 1, c)].start()

    return pl.pallas_call(
        body,
        out_shape=jax.ShapeDtypeStruct((m, n), jnp.float32),
        in_specs=[
            pl.BlockSpec(memory_space=pltpu.VMEM),
            pl.BlockSpec(memory_space=pltpu.VMEM),
        ],
        out_specs=pl.BlockSpec(memory_space=pltpu.VMEM),
        scratch_shapes=[
            pltpu.VMEM((nu, h0, wch), jnp.float32),
            pltpu.VMEM((nu, h1, wch), jnp.float32),
            pltpu.VMEM((nu, 3, zrows, wch), jnp.float32),
            pltpu.SemaphoreType.DMA((2, nu)),
            pltpu.SemaphoreType.DMA((2, nu)),
            pltpu.SemaphoreType.DMA((nu, 3)),
            pltpu.SemaphoreType.DMA((nu, 3)),
            pltpu.SemaphoreType.DMA((nu, 3)),
            pltpu.SemaphoreType.DMA((nu, 3)),
            pltpu.SemaphoreType.DMA((2, nu)),
            pltpu.SemaphoreType.DMA((2, nu)),
        ],
        compiler_params=pltpu.CompilerParams(collective_id=0),
    )(x, w_mat)
